# baseline (speedup 1.0000x reference)
import functools

import jax
import jax.numpy as jnp
from jax import lax
from jax.experimental import pallas as pl
from jax.experimental.pallas import tpu as pltpu
from jax.experimental.pallas import tpu_sc as plsc

_info = plsc.get_sparse_core_info()
_NC, _NS = _info.num_cores, _info.num_subcores
_NW = _NC * _NS

_CH = 64


def _make_sc_broadcast(B, T, D, dtype):
    rows_per_w = T // _NW
    chunks = rows_per_w // _CH
    mesh = plsc.VectorSubcoreMesh(core_axis_name="c", subcore_axis_name="s")

    @functools.partial(
        pl.kernel,
        mesh=mesh,
        out_type=jax.ShapeDtypeStruct((B, T, D), dtype),
        scratch_types=[
            pltpu.VMEM((_CH, D), dtype),
            pltpu.SemaphoreType.DMA,
        ],
    )
    def sc_broadcast(table_hbm, out_hbm, buf, sem):
        wid = lax.axis_index("s") * _NC + lax.axis_index("c")
        base = wid * rows_per_w
        for i in range(chunks):
            r0 = base + i * _CH
            pltpu.sync_copy(table_hbm.at[pl.ds(r0, _CH)], buf)
            for b in range(B):
                pltpu.async_copy(buf, out_hbm.at[b, pl.ds(r0, _CH)], sem)
            for b in range(B):
                pltpu.make_async_copy(
                    buf, out_hbm.at[b, pl.ds(r0, _CH)], sem
                ).wait()

    return sc_broadcast


def kernel(x, pos_table):
    B, T, D = x.shape
    return _make_sc_broadcast(B, T, D, pos_table.dtype)(pos_table)
